# baseline (speedup 1.0000x reference)
import jax
import jax.numpy as jnp
from jax.experimental import pallas as pl

_COLS = 65536
_C = 1024
_STEPS = _COLS // _C


def _wsum_kernel(x_ref, v_ref, o_ref):
    acc = jnp.zeros((2, _C), jnp.float32)
    for j in range(_STEPS):
        xb = x_ref[:, j * _C:(j + 1) * _C].astype(jnp.bfloat16).astype(jnp.float32)
        acc = acc + xb
    vb = v_ref[...].astype(jnp.bfloat16).astype(jnp.float32)
    rs = jnp.sum(acc, axis=1, keepdims=True)
    o_ref[...] = jnp.sum(rs * vb, axis=(0, 1), keepdims=True)


def kernel(x, values):
    out = pl.pallas_call(
        _wsum_kernel,
        out_shape=jax.ShapeDtypeStruct((1, 1), jnp.float32),
    )(x, values.reshape(2, 1))
    return out[0, 0]
